# baseline (speedup 1.0000x reference)
import functools

import jax
import jax.numpy as jnp
from jax import lax
from jax.experimental import pallas as pl
from jax.experimental.pallas import tpu as pltpu
from jax.experimental.pallas import tpu_sc as plsc

NUM_CORES = 2
NUM_SUBCORES = 16
NUM_WORKERS = NUM_CORES * NUM_SUBCORES

LANES = 16
VPC = 20
CHUNK = LANES * VPC
NBUF = 2


def _build_sc_gather(total_rows: int, features: int):
    assert total_rows % (NUM_WORKERS * CHUNK) == 0
    rows_per_w = total_rows // NUM_WORKERS
    chunks_per_w = rows_per_w // CHUNK
    n_chunks = total_rows // CHUNK

    mesh = plsc.VectorSubcoreMesh(
        core_axis_name="c", subcore_axis_name="s",
        num_cores=NUM_CORES, num_subcores=NUM_SUBCORES)

    @functools.partial(
        pl.kernel,
        out_type=jax.ShapeDtypeStruct((n_chunks, CHUNK, features),
                                      jnp.float32),
        mesh=mesh,
        scratch_types=[
            pltpu.VMEM((chunks_per_w, VPC, LANES), jnp.int32),
            pltpu.VMEM((NBUF, CHUNK, features), jnp.float32),
            [pltpu.SemaphoreType.DMA] * NBUF,
            [pltpu.SemaphoreType.DMA] * NBUF,
        ],
        compiler_params=pltpu.CompilerParams(use_tc_tiling_on_sc=False),
    )
    def sc_gather(idx_hbm, tab_hbm, out_hbm, idx_v, buf, gsems, wsems):
        wid = lax.axis_index("s") * NUM_CORES + lax.axis_index("c")
        chunk0 = wid * chunks_per_w
        pltpu.sync_copy(idx_hbm.at[wid], idx_v)

        def fire_chunk(j, b):
            for u in range(VPC):
                iv = idx_v[j, u]
                pltpu.async_copy(
                    tab_hbm.at[iv], buf.at[b, pl.ds(u * LANES, LANES)],
                    gsems[b])

        def drain_chunk(b):
            for u in range(VPC):
                pltpu.make_async_copy(
                    tab_hbm.at[idx_v[0, 0]],
                    buf.at[b, pl.ds(u * LANES, LANES)], gsems[b]).wait()

        def wait_write(b):
            pltpu.make_async_copy(buf.at[b], out_hbm.at[chunk0],
                                  wsems[b]).wait()

        for b in range(NBUF):
            fire_chunk(b, b)

        @pl.loop(0, chunks_per_w // NBUF)
        def _(jo):
            j0 = jo * NBUF
            for b in range(NBUF):
                drain_chunk(b)
                pltpu.async_copy(buf.at[b], out_hbm.at[chunk0 + j0 + b],
                                 wsems[b])
            for b in range(NBUF):
                nj = j0 + NBUF + b

                @pl.when(nj < chunks_per_w)
                def _():
                    wait_write(b)
                    fire_chunk(nj, b)

        for b in range(NBUF):
            wait_write(b)

    return sc_gather


def kernel(inputs, embedding):
    b, s = inputs.shape
    total = b * s
    idx4d = inputs.reshape(NUM_WORKERS, -1, VPC, LANES).astype(jnp.int32)
    gather = _build_sc_gather(total, embedding.shape[1])
    out = gather(idx4d, embedding)
    return out.reshape(b, s, embedding.shape[1])
